# Initial kernel scaffold; baseline (speedup 1.0000x reference)
import jax
import jax.numpy as jnp
from jax.experimental import pallas as pl


def kernel(g, h, i, j, W, b):
    raise NotImplementedError("write your pallas kernel here")


# speedup vs baseline: 6.3572x; 6.3572x over previous
import jax
import jax.numpy as jnp
from jax import lax
from jax.experimental import pallas as pl
from jax.experimental.pallas import tpu as pltpu
from jax.experimental.pallas import tpu_sc as plsc

_N = 10000
_E = 320000
_D = 128
_L = 16
_NS = 16
_EPW = _E // _NS
_ITERS = _EPW // _L


def _matvec_body(h_ref, w_ref, o_ref):
    o_ref[...] = jnp.dot(h_ref[...], w_ref[...],
                         preferred_element_type=jnp.float32)


def _edge_scan_body(u_hbm, g_hbm, pi_hbm, pj_hbm, pb_hbm, out_hbm,
                    u_v, g_v, pi_v, pj_v, pb_v, stage_v, parts_sh, parts_v):
    s = lax.axis_index("s")
    pltpu.sync_copy(pi_hbm, pi_v)
    pltpu.sync_copy(pj_hbm, pj_v)
    pltpu.sync_copy(pb_hbm, pb_v)
    pltpu.sync_copy(u_hbm, u_v)
    pltpu.sync_copy(g_hbm.at[pl.ds(s * (_EPW * 2), _EPW * 2)], g_v)
    pi = pi_v[...]
    bb = pb_v[...]
    lanes = lax.iota(jnp.int32, _L)
    si = plsc.load_gather(u_v, [2 * pi + 1])
    sib = si + bb

    def body(k, acc):
        e0 = k * (2 * _L) + 2 * lanes
        src = plsc.load_gather(g_v, [e0])
        dst = plsc.load_gather(g_v, [e0 + 1])
        tv = plsc.load_gather(u_v, [2 * dst])
        x = sib + tv
        lr = jnp.where(x >= 0, x, 0.2 * x)
        return acc + jnp.where(src == pi, lr, 0.0)

    acc = lax.fori_loop(0, _ITERS, body, jnp.zeros((_L,), jnp.float32))
    stage_v[...] = acc
    pltpu.sync_copy(stage_v, parts_sh.at[s])
    plsc.subcore_barrier()

    @pl.when(s == 0)
    def _():
        pltpu.sync_copy(parts_sh, parts_v)
        tot16 = jnp.zeros((_L,), jnp.float32)
        for r in range(_NS):
            tot16 = tot16 + parts_v[r]
        total = jnp.sum(tot16)
        pj = pj_v[...]
        tj = plsc.load_gather(u_v, [2 * pj])
        x0 = sib + tj
        eij = jnp.where(x0 >= 0, x0, 0.2 * x0)
        stage_v[...] = eij / total
        pltpu.sync_copy(stage_v, out_hbm)


def kernel(g, h, i, j, W, b):
    Wl = W[0, :_D]
    Wr = W[0, _D:]
    wcat = jnp.stack([Wr, Wl], axis=1)
    u = pl.pallas_call(
        _matvec_body,
        out_shape=jax.ShapeDtypeStruct((_N, 2), jnp.float32),
    )(h, wcat)
    uflat = u.reshape(_N * 2)
    gflat = g.reshape(_E * 2)
    pi = jnp.full((_L,), jnp.asarray(i, jnp.int32), jnp.int32)
    pj = jnp.full((_L,), jnp.asarray(j, jnp.int32), jnp.int32)
    pb = jnp.full((_L,), b[0], jnp.float32)
    mesh = plsc.VectorSubcoreMesh(core_axis_name="c", subcore_axis_name="s",
                                  num_cores=2, num_subcores=_NS)
    scan = pl.kernel(
        _edge_scan_body,
        out_type=jax.ShapeDtypeStruct((_L,), jnp.float32),
        mesh=mesh,
        compiler_params=pltpu.CompilerParams(
            needs_layout_passes=False, use_tc_tiling_on_sc=False),
        scratch_types=[
            pltpu.VMEM((_N * 2,), jnp.float32),
            pltpu.VMEM((_EPW * 2,), jnp.int32),
            pltpu.VMEM((_L,), jnp.int32),
            pltpu.VMEM((_L,), jnp.int32),
            pltpu.VMEM((_L,), jnp.float32),
            pltpu.VMEM((_L,), jnp.float32),
            pltpu.VMEM_SHARED((_NS, _L), jnp.float32),
            pltpu.VMEM((_NS, _L), jnp.float32),
        ],
    )(uflat, gflat, pi, pj, pb)
    return scan[0:1]
